# baseline (speedup 1.0000x reference)
import functools

import jax
import jax.numpy as jnp
from jax import lax
from jax.experimental import pallas as pl
from jax.experimental.pallas import tpu as pltpu
from jax.experimental.pallas import tpu_sc as plsc

NUM_BUCKETS = 1000000
EMBED_DIM = 32
BATCH = 4
SEQ_LEN = 8192
FLAT = BATCH * SEQ_LEN

A_HI = 761000
A_LO = 435761
B_HI = 503000
B_LO = 40503

_INFO = plsc.get_sparse_core_info()
NC = _INFO.num_cores
NS = _INFO.num_subcores
L = _INFO.num_lanes
NW = NC * NS
CHUNK = FLAT // NW
STEPS = CHUNK // L
GATHER_W = 128
NGATHER = CHUNK // GATHER_W


def _sc_body(ids_hbm, table_hbm, out_hbm, ids_v, idx_v, rows_v, sem):
    wid = lax.axis_index("s") * NC + lax.axis_index("c")
    base = wid * CHUNK

    zeros = jnp.zeros((L,), jnp.int32)
    lane = lax.iota(jnp.int32, L)
    plsc.store_scatter(ids_v, [lane], zeros)
    pltpu.sync_copy(ids_hbm.at[pl.ds(base, CHUNK)], ids_v.at[pl.ds(8, CHUNK)])

    @pl.when(wid % (NW // BATCH) != 0)
    def _():
        pltpu.sync_copy(ids_hbm.at[pl.ds(base - 8, 8)], ids_v.at[pl.ds(0, 8)])

    a_hi = jnp.int32(A_HI)
    a_lo = jnp.int32(A_LO)
    b_hi = jnp.int32(B_HI)
    b_lo = jnp.int32(B_LO)
    thousand = jnp.int32(1000)
    nbuckets = jnp.int32(NUM_BUCKETS)

    def hash_step(_, off):
        cur = plsc.load_gather(ids_v, [lane + (off + jnp.int32(8))])
        prev = plsc.load_gather(ids_v, [lane + (off + jnp.int32(7))])
        p1 = prev // thousand
        p0 = prev - p1 * thousand
        c1 = cur // thousand
        c0 = cur - c1 * thousand
        h = (a_hi * p1 + a_lo * p0 + b_hi * c1 + b_lo * c0) % nbuckets
        plsc.store_scatter(idx_v, [lane + off], h)
        return off + jnp.int32(L)

    lax.fori_loop(0, STEPS, hash_step, jnp.int32(0))

    copies = []
    for j in range(NGATHER):
        copies.append(pltpu.async_copy(
            table_hbm.at[idx_v.at[pl.ds(j * GATHER_W, GATHER_W)]],
            rows_v.at[pl.ds(j * GATHER_W, GATHER_W)],
            sem))
    for c in copies:
        c.wait()

    pltpu.sync_copy(rows_v, out_hbm.at[pl.ds(base, CHUNK)])


@jax.jit
def _bigram_embed(ids_flat, table):
    mesh = plsc.VectorSubcoreMesh(core_axis_name="c", subcore_axis_name="s")
    run = functools.partial(
        pl.kernel,
        out_type=jax.ShapeDtypeStruct((FLAT, EMBED_DIM), jnp.float32),
        mesh=mesh,
        scratch_types=[
            pltpu.VMEM((CHUNK + 16,), jnp.int32),
            pltpu.VMEM((CHUNK,), jnp.int32),
            pltpu.VMEM((CHUNK, EMBED_DIM), jnp.float32),
            pltpu.SemaphoreType.DMA,
        ],
        compiler_params=pltpu.CompilerParams(
            needs_layout_passes=False, use_tc_tiling_on_sc=False),
    )(_sc_body)
    return run(ids_flat, table)


def kernel(input_ids, embedding_weight):
    ids_flat = input_ids.reshape(-1).astype(jnp.int32)
    out = _bigram_embed(ids_flat, embedding_weight)
    return out.reshape(BATCH, SEQ_LEN, EMBED_DIM)
